# baseline (speedup 1.0000x reference)
import functools

import jax
import jax.numpy as jnp
from jax import lax
from jax.experimental import pallas as pl
from jax.experimental.pallas import tpu as pltpu
from jax.experimental.pallas import tpu_sc as plsc

_NW = 32
_BUF = 64
_ROWS = 8192
_D = 1024
_PER_W = _ROWS // _NW


def _make_sc_kernel():
    mesh = plsc.VectorSubcoreMesh(core_axis_name="c", subcore_axis_name="s")

    @functools.partial(
        pl.kernel,
        mesh=mesh,
        out_type=jax.ShapeDtypeStruct((_ROWS, _D), jnp.float32),
        scratch_types=[
            pltpu.VMEM((_BUF,), jnp.int32),
            pltpu.VMEM((_BUF, _D), jnp.float32),
            pltpu.SemaphoreType.DMA,
        ],
    )
    def sc_broadcast(table_hbm, idx_hbm, out_hbm, idx_v, buf, sem):
        wid = lax.axis_index("s") * 2 + lax.axis_index("c")
        base = wid * _PER_W
        pltpu.sync_copy(idx_hbm, idx_v)
        pltpu.async_copy(table_hbm.at[idx_v], buf, sem).wait()
        cps = [
            pltpu.async_copy(buf, out_hbm.at[pl.ds(base + j * _BUF, _BUF)], sem)
            for j in range(_PER_W // _BUF)
        ]
        for c in cps:
            c.wait()

    return sc_broadcast


_sc_kernel = _make_sc_kernel()


def kernel(inputs, num_particle, lookup_table):
    B, T, D = inputs.shape
    idx = jnp.full((_BUF,), jnp.asarray(num_particle, jnp.int32), dtype=jnp.int32)
    out = _sc_kernel(lookup_table, idx)
    return out.reshape(B, T, D)


# speedup vs baseline: 9.9752x; 9.9752x over previous
import jax
import jax.numpy as jnp
from jax.experimental import pallas as pl
from jax.experimental.pallas import tpu as pltpu

_CH = 128


def _body(idx_ref, table_ref, out_ref, scratch, sem):
    r = idx_ref[0] % 8
    scratch[...] = jnp.broadcast_to(table_ref[pl.ds(r, 1), :], scratch.shape)
    n = out_ref.shape[0] // _CH
    copies = [
        pltpu.make_async_copy(scratch, out_ref.at[pl.ds(k * _CH, _CH), :], sem.at[k % 4])
        for k in range(n)
    ]
    for c in copies:
        c.start()
    for c in copies:
        c.wait()


def kernel(inputs, num_particle, lookup_table):
    B, T, D = inputs.shape
    rows = B * T
    idx = jnp.asarray(num_particle, jnp.int32).reshape(1)
    out = pl.pallas_call(
        _body,
        grid_spec=pltpu.PrefetchScalarGridSpec(
            num_scalar_prefetch=1,
            grid=(1,),
            in_specs=[pl.BlockSpec((8, D), lambda i, idx_ref: (idx_ref[0] // 8, 0))],
            out_specs=pl.BlockSpec(memory_space=pltpu.MemorySpace.HBM),
            scratch_shapes=[
                pltpu.VMEM((_CH, D), jnp.float32),
                pltpu.SemaphoreType.DMA((4,)),
            ],
        ),
        out_shape=jax.ShapeDtypeStruct((rows, D), jnp.float32),
    )(idx, lookup_table)
    return out.reshape(B, T, D)


# speedup vs baseline: 9.9850x; 1.0010x over previous
import jax
import jax.numpy as jnp
from jax.experimental import pallas as pl
from jax.experimental.pallas import tpu as pltpu

_CH = 128


def _body(idx_ref, table_ref, out_ref, scratch, sem):
    r = idx_ref[0] % 8
    scratch[...] = jnp.broadcast_to(table_ref[pl.ds(r, 1), :], scratch.shape)
    n = out_ref.shape[0] // _CH
    copies = [
        pltpu.make_async_copy(scratch, out_ref.at[pl.ds(k * _CH, _CH), :], sem)
        for k in range(n)
    ]
    for c in copies:
        c.start()
    for c in copies:
        c.wait()


def kernel(inputs, num_particle, lookup_table):
    B, T, D = inputs.shape
    rows = B * T
    idx = jnp.asarray(num_particle, jnp.int32).reshape(1)
    out = pl.pallas_call(
        _body,
        grid_spec=pltpu.PrefetchScalarGridSpec(
            num_scalar_prefetch=1,
            grid=(1,),
            in_specs=[pl.BlockSpec((8, D), lambda i, idx_ref: (idx_ref[0] // 8, 0))],
            out_specs=pl.BlockSpec(memory_space=pltpu.MemorySpace.HBM),
            scratch_shapes=[
                pltpu.VMEM((_CH, D), jnp.float32),
                pltpu.SemaphoreType.DMA,
            ],
        ),
        out_shape=jax.ShapeDtypeStruct((rows, D), jnp.float32),
    )(idx, lookup_table)
    return out.reshape(B, T, D)
